# Initial kernel scaffold; baseline (speedup 1.0000x reference)
import jax
import jax.numpy as jnp
from jax.experimental import pallas as pl


def kernel(batch_box_preds, batch_cls_preds, batch_dir_preds, batch_anchors):
    raise NotImplementedError("write your pallas kernel here")


# speedup vs baseline: 7.7480x; 7.7480x over previous
import jax
import jax.numpy as jnp
from jax.experimental import pallas as pl
from jax.experimental.pallas import tpu as pltpu

_NUM_CLASS = 2
_NMS_PRE = 1000
_NMS_POST = 300
_IOU_TH = 0.5
_NCAND = 1024
_NOUT = 384
_CHUNK = 1000
_PERIOD = 3.141592653589793


def _scores_body(cls_ref, out_ref):
    c = cls_ref[...]
    c0 = c[0:1, :]
    c1 = c[1:2, :]
    m = jnp.maximum(c0, c1)
    e0 = jnp.exp(c0 - m)
    e1 = jnp.exp(c1 - m)
    out_ref[...] = e1 / (e0 + e1)


def _nms_body(data_ref, idx_ref, sc_ref, out_ref, sup_ref):
    n = data_ref.shape[0]
    idx = idx_ref[...]
    nch = n // _CHUNK

    acc = jnp.zeros((_NCAND, 16), jnp.float32)
    colbase = jax.lax.broadcasted_iota(jnp.int32, (_NCAND, _CHUNK), 1)
    for c in range(nch):
        oh = (idx == (colbase + c * _CHUNK)).astype(jnp.float32)
        acc = acc + jnp.dot(oh, data_ref[c * _CHUNK:(c + 1) * _CHUNK, :],
                            preferred_element_type=jnp.float32, precision=jax.lax.Precision.HIGHEST)
    g = acc

    xt, yt, zt = g[:, 0:1], g[:, 1:2], g[:, 2:3]
    wt, lt, ht, rt = g[:, 3:4], g[:, 4:5], g[:, 5:6], g[:, 6:7]
    xa, ya, za = g[:, 7:8], g[:, 8:9], g[:, 9:10]
    wa, la, ha, ra = g[:, 10:11], g[:, 11:12], g[:, 12:13], g[:, 13:14]
    d0, d1 = g[:, 14:15], g[:, 15:16]

    diag = jnp.sqrt(la * la + wa * wa)
    xg = xt * diag + xa
    yg = yt * diag + ya
    zg = zt * ha + za
    wg = jnp.exp(wt) * wa
    lg = jnp.exp(lt) * la
    hg = jnp.exp(ht) * ha
    rg = rt + ra
    dirf = (d1 > d0).astype(jnp.float32)

    cr = jnp.abs(jnp.cos(rg))
    sr = jnp.abs(jnp.sin(rg))
    ex = (cr * wg + sr * lg) * 0.5
    ey = (sr * wg + cr * lg) * 0.5
    x1 = xg - ex
    x2 = xg + ex
    y1 = yg - ey
    y2 = yg + ey
    area = (x2 - x1) * (y2 - y1)

    scat = jnp.concatenate([x1, y1, x2, y2, area, jnp.zeros((_NCAND, 3), jnp.float32)], axis=1)
    st = scat.T
    x1r, y1r = st[0:1, :], st[1:2, :]
    x2r, y2r = st[2:3, :], st[3:4, :]
    arear = st[4:5, :]

    rb = 256
    for r0 in range(0, _NCAND, rb):
        x1b, y1b = x1[r0:r0 + rb], y1[r0:r0 + rb]
        x2b, y2b = x2[r0:r0 + rb], y2[r0:r0 + rb]
        areab = area[r0:r0 + rb]
        ix1 = jnp.maximum(x1b, x1r)
        iy1 = jnp.maximum(y1b, y1r)
        ix2 = jnp.minimum(x2b, x2r)
        iy2 = jnp.minimum(y2b, y2r)
        iw = jnp.clip(ix2 - ix1, 0.0)
        ih = jnp.clip(iy2 - iy1, 0.0)
        inter = iw * ih
        iou = inter / (areab + arear - inter + 1e-6)
        rowi = jax.lax.broadcasted_iota(jnp.int32, (rb, _NCAND), 0) + r0
        coli = jax.lax.broadcasted_iota(jnp.int32, (rb, _NCAND), 1)
        sup = (iou > _IOU_TH) & (rowi < coli) & (rowi < _NMS_PRE) & (coli < _NMS_PRE)
        sup_ref[r0:r0 + rb, :] = sup.astype(jnp.float32)

    def cond(carry):
        return carry[1]

    def body(carry):
        k, _ = carry
        s = jnp.dot(k, sup_ref[...], preferred_element_type=jnp.float32, precision=jax.lax.Precision.HIGHEST)
        newk = jnp.where(s > 0.0, 0.0, 1.0)
        return newk, jnp.any(newk != k)

    k0 = jnp.ones((1, _NCAND), jnp.float32)
    kfin, _ = jax.lax.while_loop(cond, body, (k0, jnp.bool_(True)))

    colv = jax.lax.broadcasted_iota(jnp.int32, (1, _NCAND), 1)
    kept = jnp.where(colv < _NMS_PRE, kfin, 0.0)

    ltr = jax.lax.broadcasted_iota(jnp.int32, (_NCAND, _NCAND), 0)
    ltc = jax.lax.broadcasted_iota(jnp.int32, (_NCAND, _NCAND), 1)
    lt = (ltr <= ltc).astype(jnp.float32)
    pos = jnp.dot(kept, lt, preferred_element_type=jnp.float32, precision=jax.lax.Precision.HIGHEST)

    orow = jax.lax.broadcasted_iota(jnp.int32, (_NOUT, 1), 0).astype(jnp.float32) + 1.0
    m = ((pos == orow) & (kept > 0.0)).astype(jnp.float32)

    r_adj = rg - jnp.floor(rg / _PERIOD) * _PERIOD + _PERIOD * dirf
    cok = ((xg >= 0.0) & (xg <= 70.4) & (yg >= -40.0) & (yg <= 40.0)
           & (zg >= -3.0) & (zg <= 1.0)).astype(jnp.float32)
    one = jnp.ones((_NCAND, 1), jnp.float32)
    d2 = jnp.concatenate(
        [xg, yg, zg, wg, lg, hg, r_adj, sc_ref[...], cok, one,
         jnp.zeros((_NCAND, 6), jnp.float32)], axis=1)
    sel = jnp.dot(m, d2, preferred_element_type=jnp.float32, precision=jax.lax.Precision.HIGHEST)

    filled = sel[:, 9:10] > 0.5
    cokb = sel[:, 8:9] > 0.5
    validb = filled & cokb
    boxes = jnp.where(validb, sel[:, 0:7], 0.0)
    scoreo = jnp.where(validb, sel[:, 7:8], 0.0)
    out_ref[...] = jnp.concatenate(
        [boxes, scoreo, validb.astype(jnp.float32), jnp.zeros((_NOUT, 7), jnp.float32)], axis=1)


def kernel(batch_box_preds, batch_cls_preds, batch_dir_preds, batch_anchors):
    b, n = batch_cls_preds.shape[0], batch_cls_preds.shape[1]
    cls_t = jnp.swapaxes(batch_cls_preds, 1, 2)
    scores = pl.pallas_call(
        _scores_body,
        grid=(b,),
        in_specs=[pl.BlockSpec((None, _NUM_CLASS, n), lambda i: (i, 0, 0))],
        out_specs=pl.BlockSpec((None, 1, n), lambda i: (i, 0, 0)),
        out_shape=jax.ShapeDtypeStruct((b, 1, n), jnp.float32),
    )(cls_t).reshape(b, n)

    sc, idx = jax.lax.top_k(scores, _NMS_PRE)
    idxp = jnp.pad(idx.astype(jnp.int32), ((0, 0), (0, _NCAND - _NMS_PRE)))[..., None]
    scp = jnp.pad(sc, ((0, 0), (0, _NCAND - _NMS_PRE)))[..., None]

    data = jnp.concatenate([batch_box_preds, batch_anchors, batch_dir_preds], axis=-1)

    out = pl.pallas_call(
        _nms_body,
        grid=(b,),
        in_specs=[
            pl.BlockSpec((None, n, 16), lambda i: (i, 0, 0)),
            pl.BlockSpec((None, _NCAND, 1), lambda i: (i, 0, 0)),
            pl.BlockSpec((None, _NCAND, 1), lambda i: (i, 0, 0)),
        ],
        out_specs=pl.BlockSpec((None, _NOUT, 16), lambda i: (i, 0, 0)),
        out_shape=jax.ShapeDtypeStruct((b, _NOUT, 16), jnp.float32),
        scratch_shapes=[pltpu.VMEM((_NCAND, _NCAND), jnp.float32)],
    )(data, idxp, scp)

    final_boxes = out[:, :_NMS_POST, :7]
    final_scores = out[:, :_NMS_POST, 7]
    final_labels = jnp.zeros((b, _NMS_POST), jnp.int32)
    valid = out[:, :_NMS_POST, 8] > 0.5
    return final_boxes, final_scores, final_labels, valid


# speedup vs baseline: 7.7589x; 1.0014x over previous
import jax
import jax.numpy as jnp
from jax.experimental import pallas as pl
from jax.experimental.pallas import tpu as pltpu

_NUM_CLASS = 2
_NMS_PRE = 1000
_NMS_POST = 300
_IOU_TH = 0.5
_NCAND = 1024
_NOUT = 384
_CHUNK = 1000
_PERIOD = 3.141592653589793


def _scores_body(cls_ref, out_ref):
    c = cls_ref[...]
    c0 = c[0:1, :]
    c1 = c[1:2, :]
    m = jnp.maximum(c0, c1)
    e0 = jnp.exp(c0 - m)
    e1 = jnp.exp(c1 - m)
    out_ref[...] = e1 / (e0 + e1)


def _nms_body(data_ref, idx_ref, sc_ref, out_ref, sup_ref):
    n = data_ref.shape[0]
    idx = idx_ref[...]
    nch = n // _CHUNK

    idx_lo = idx % _CHUNK
    idx_hi = idx // _CHUNK
    colbase = jax.lax.broadcasted_iota(jnp.int32, (_NCAND, _CHUNK), 1)
    oh = (idx_lo == colbase).astype(jnp.float32)
    acc = jnp.zeros((_NCAND, 16), jnp.float32)
    for c in range(nch):
        part = jnp.dot(oh, data_ref[c * _CHUNK:(c + 1) * _CHUNK, :],
                       preferred_element_type=jnp.float32, precision=jax.lax.Precision.HIGHEST)
        acc = acc + jnp.where(idx_hi == c, part, 0.0)
    g = acc

    xt, yt, zt = g[:, 0:1], g[:, 1:2], g[:, 2:3]
    wt, lt, ht, rt = g[:, 3:4], g[:, 4:5], g[:, 5:6], g[:, 6:7]
    xa, ya, za = g[:, 7:8], g[:, 8:9], g[:, 9:10]
    wa, la, ha, ra = g[:, 10:11], g[:, 11:12], g[:, 12:13], g[:, 13:14]
    d0, d1 = g[:, 14:15], g[:, 15:16]

    diag = jnp.sqrt(la * la + wa * wa)
    xg = xt * diag + xa
    yg = yt * diag + ya
    zg = zt * ha + za
    wg = jnp.exp(wt) * wa
    lg = jnp.exp(lt) * la
    hg = jnp.exp(ht) * ha
    rg = rt + ra
    dirf = (d1 > d0).astype(jnp.float32)

    cr = jnp.abs(jnp.cos(rg))
    sr = jnp.abs(jnp.sin(rg))
    ex = (cr * wg + sr * lg) * 0.5
    ey = (sr * wg + cr * lg) * 0.5
    x1 = xg - ex
    x2 = xg + ex
    y1 = yg - ey
    y2 = yg + ey
    area = (x2 - x1) * (y2 - y1)

    scat = jnp.concatenate([x1, y1, x2, y2, area, jnp.zeros((_NCAND, 3), jnp.float32)], axis=1)
    st = scat.T
    x1r, y1r = st[0:1, :], st[1:2, :]
    x2r, y2r = st[2:3, :], st[3:4, :]
    arear = st[4:5, :]

    rb = 256
    for r0 in range(0, _NCAND, rb):
        x1b, y1b = x1[r0:r0 + rb], y1[r0:r0 + rb]
        x2b, y2b = x2[r0:r0 + rb], y2[r0:r0 + rb]
        areab = area[r0:r0 + rb]
        ix1 = jnp.maximum(x1b, x1r)
        iy1 = jnp.maximum(y1b, y1r)
        ix2 = jnp.minimum(x2b, x2r)
        iy2 = jnp.minimum(y2b, y2r)
        iw = jnp.clip(ix2 - ix1, 0.0)
        ih = jnp.clip(iy2 - iy1, 0.0)
        inter = iw * ih
        iou = inter / (areab + arear - inter + 1e-6)
        rowi = jax.lax.broadcasted_iota(jnp.int32, (rb, _NCAND), 0) + r0
        coli = jax.lax.broadcasted_iota(jnp.int32, (rb, _NCAND), 1)
        sup = (iou > _IOU_TH) & (rowi < coli) & (rowi < _NMS_PRE) & (coli < _NMS_PRE)
        sup_ref[r0:r0 + rb, :] = sup.astype(jnp.float32)

    def cond(carry):
        return carry[1]

    def body(carry):
        k, _ = carry
        s = jnp.dot(k, sup_ref[...], preferred_element_type=jnp.float32, precision=jax.lax.Precision.HIGHEST)
        newk = jnp.where(s > 0.0, 0.0, 1.0)
        return newk, jnp.any(newk != k)

    k0 = jnp.ones((1, _NCAND), jnp.float32)
    kfin, _ = jax.lax.while_loop(cond, body, (k0, jnp.bool_(True)))

    colv = jax.lax.broadcasted_iota(jnp.int32, (1, _NCAND), 1)
    kept = jnp.where(colv < _NMS_PRE, kfin, 0.0)

    ltr = jax.lax.broadcasted_iota(jnp.int32, (_NCAND, _NCAND), 0)
    ltc = jax.lax.broadcasted_iota(jnp.int32, (_NCAND, _NCAND), 1)
    lt = (ltr <= ltc).astype(jnp.float32)
    pos = jnp.dot(kept, lt, preferred_element_type=jnp.float32, precision=jax.lax.Precision.HIGHEST)

    orow = jax.lax.broadcasted_iota(jnp.int32, (_NOUT, 1), 0).astype(jnp.float32) + 1.0
    m = ((pos == orow) & (kept > 0.0)).astype(jnp.float32)

    r_adj = rg - jnp.floor(rg / _PERIOD) * _PERIOD + _PERIOD * dirf
    cok = ((xg >= 0.0) & (xg <= 70.4) & (yg >= -40.0) & (yg <= 40.0)
           & (zg >= -3.0) & (zg <= 1.0)).astype(jnp.float32)
    one = jnp.ones((_NCAND, 1), jnp.float32)
    d2 = jnp.concatenate(
        [xg, yg, zg, wg, lg, hg, r_adj, sc_ref[...], cok, one,
         jnp.zeros((_NCAND, 6), jnp.float32)], axis=1)
    sel = jnp.dot(m, d2, preferred_element_type=jnp.float32, precision=jax.lax.Precision.HIGHEST)

    filled = sel[:, 9:10] > 0.5
    cokb = sel[:, 8:9] > 0.5
    validb = filled & cokb
    boxes = jnp.where(validb, sel[:, 0:7], 0.0)
    scoreo = jnp.where(validb, sel[:, 7:8], 0.0)
    out_ref[...] = jnp.concatenate(
        [boxes, scoreo, validb.astype(jnp.float32), jnp.zeros((_NOUT, 7), jnp.float32)], axis=1)


def kernel(batch_box_preds, batch_cls_preds, batch_dir_preds, batch_anchors):
    b, n = batch_cls_preds.shape[0], batch_cls_preds.shape[1]
    cls_t = jnp.swapaxes(batch_cls_preds, 1, 2)
    scores = pl.pallas_call(
        _scores_body,
        grid=(b,),
        in_specs=[pl.BlockSpec((None, _NUM_CLASS, n), lambda i: (i, 0, 0))],
        out_specs=pl.BlockSpec((None, 1, n), lambda i: (i, 0, 0)),
        out_shape=jax.ShapeDtypeStruct((b, 1, n), jnp.float32),
    )(cls_t).reshape(b, n)

    sc, idx = jax.lax.top_k(scores, _NMS_PRE)
    idxp = jnp.pad(idx.astype(jnp.int32), ((0, 0), (0, _NCAND - _NMS_PRE)))[..., None]
    scp = jnp.pad(sc, ((0, 0), (0, _NCAND - _NMS_PRE)))[..., None]

    data = jnp.concatenate([batch_box_preds, batch_anchors, batch_dir_preds], axis=-1)

    out = pl.pallas_call(
        _nms_body,
        grid=(b,),
        in_specs=[
            pl.BlockSpec((None, n, 16), lambda i: (i, 0, 0)),
            pl.BlockSpec((None, _NCAND, 1), lambda i: (i, 0, 0)),
            pl.BlockSpec((None, _NCAND, 1), lambda i: (i, 0, 0)),
        ],
        out_specs=pl.BlockSpec((None, _NOUT, 16), lambda i: (i, 0, 0)),
        out_shape=jax.ShapeDtypeStruct((b, _NOUT, 16), jnp.float32),
        scratch_shapes=[pltpu.VMEM((_NCAND, _NCAND), jnp.float32)],
    )(data, idxp, scp)

    final_boxes = out[:, :_NMS_POST, :7]
    final_scores = out[:, :_NMS_POST, 7]
    final_labels = jnp.zeros((b, _NMS_POST), jnp.int32)
    valid = out[:, :_NMS_POST, 8] > 0.5
    return final_boxes, final_scores, final_labels, valid


# speedup vs baseline: 12.3178x; 1.5876x over previous
import jax
import jax.numpy as jnp
from jax.experimental import pallas as pl
from jax.experimental.pallas import tpu as pltpu

_NUM_CLASS = 2
_NMS_PRE = 1000
_NMS_POST = 300
_IOU_TH = 0.5
_NCAND = 1024
_NOUT = 384
_CHUNK = 1000
_PERIOD = 3.141592653589793


def _scores_body(cls_ref, out_ref):
    c = cls_ref[...]
    c0 = c[0:1, :]
    c1 = c[1:2, :]
    m = jnp.maximum(c0, c1)
    e0 = jnp.exp(c0 - m)
    e1 = jnp.exp(c1 - m)
    out_ref[...] = e1 / (e0 + e1)


def _nms_body(data_ref, idx_ref, sc_ref, out_ref, sup_ref, g_ref):
    b = pl.program_id(0)

    def gather_step(k, _):
        g_ref[pl.ds(k, 1), :] = data_ref[pl.ds(idx_ref[b, k], 1), :]
        return 0
    jax.lax.fori_loop(0, _NCAND, gather_step, 0, unroll=8)
    g = g_ref[...]

    xt, yt, zt = g[:, 0:1], g[:, 1:2], g[:, 2:3]
    wt, lt, ht, rt = g[:, 3:4], g[:, 4:5], g[:, 5:6], g[:, 6:7]
    xa, ya, za = g[:, 7:8], g[:, 8:9], g[:, 9:10]
    wa, la, ha, ra = g[:, 10:11], g[:, 11:12], g[:, 12:13], g[:, 13:14]
    d0, d1 = g[:, 14:15], g[:, 15:16]

    diag = jnp.sqrt(la * la + wa * wa)
    xg = xt * diag + xa
    yg = yt * diag + ya
    zg = zt * ha + za
    wg = jnp.exp(wt) * wa
    lg = jnp.exp(lt) * la
    hg = jnp.exp(ht) * ha
    rg = rt + ra
    dirf = (d1 > d0).astype(jnp.float32)

    cr = jnp.abs(jnp.cos(rg))
    sr = jnp.abs(jnp.sin(rg))
    ex = (cr * wg + sr * lg) * 0.5
    ey = (sr * wg + cr * lg) * 0.5
    x1 = xg - ex
    x2 = xg + ex
    y1 = yg - ey
    y2 = yg + ey
    area = (x2 - x1) * (y2 - y1)

    scat = jnp.concatenate([x1, y1, x2, y2, area, jnp.zeros((_NCAND, 3), jnp.float32)], axis=1)
    st = scat.T
    x1r, y1r = st[0:1, :], st[1:2, :]
    x2r, y2r = st[2:3, :], st[3:4, :]
    arear = st[4:5, :]

    rb = 256
    for r0 in range(0, _NCAND, rb):
        x1b, y1b = x1[r0:r0 + rb], y1[r0:r0 + rb]
        x2b, y2b = x2[r0:r0 + rb], y2[r0:r0 + rb]
        areab = area[r0:r0 + rb]
        ix1 = jnp.maximum(x1b, x1r)
        iy1 = jnp.maximum(y1b, y1r)
        ix2 = jnp.minimum(x2b, x2r)
        iy2 = jnp.minimum(y2b, y2r)
        iw = jnp.clip(ix2 - ix1, 0.0)
        ih = jnp.clip(iy2 - iy1, 0.0)
        inter = iw * ih
        iou = inter / (areab + arear - inter + 1e-6)
        rowi = jax.lax.broadcasted_iota(jnp.int32, (rb, _NCAND), 0) + r0
        coli = jax.lax.broadcasted_iota(jnp.int32, (rb, _NCAND), 1)
        sup = (iou > _IOU_TH) & (rowi < coli) & (rowi < _NMS_PRE) & (coli < _NMS_PRE)
        sup_ref[r0:r0 + rb, :] = sup.astype(jnp.float32)

    def cond(carry):
        return carry[1]

    def body(carry):
        k, _ = carry
        s = jnp.dot(k, sup_ref[...], preferred_element_type=jnp.float32, precision=jax.lax.Precision.HIGHEST)
        newk = jnp.where(s > 0.0, 0.0, 1.0)
        return newk, jnp.any(newk != k)

    k0 = jnp.ones((1, _NCAND), jnp.float32)
    kfin, _ = jax.lax.while_loop(cond, body, (k0, jnp.bool_(True)))

    colv = jax.lax.broadcasted_iota(jnp.int32, (1, _NCAND), 1)
    kept = jnp.where(colv < _NMS_PRE, kfin, 0.0)

    ltr = jax.lax.broadcasted_iota(jnp.int32, (_NCAND, _NCAND), 0)
    ltc = jax.lax.broadcasted_iota(jnp.int32, (_NCAND, _NCAND), 1)
    lt = (ltr <= ltc).astype(jnp.float32)
    pos = jnp.dot(kept, lt, preferred_element_type=jnp.float32, precision=jax.lax.Precision.HIGHEST)

    orow = jax.lax.broadcasted_iota(jnp.int32, (_NOUT, 1), 0).astype(jnp.float32) + 1.0
    m = ((pos == orow) & (kept > 0.0)).astype(jnp.float32)

    r_adj = rg - jnp.floor(rg / _PERIOD) * _PERIOD + _PERIOD * dirf
    cok = ((xg >= 0.0) & (xg <= 70.4) & (yg >= -40.0) & (yg <= 40.0)
           & (zg >= -3.0) & (zg <= 1.0)).astype(jnp.float32)
    one = jnp.ones((_NCAND, 1), jnp.float32)
    d2 = jnp.concatenate(
        [xg, yg, zg, wg, lg, hg, r_adj, sc_ref[...], cok, one,
         jnp.zeros((_NCAND, 6), jnp.float32)], axis=1)
    sel = jnp.dot(m, d2, preferred_element_type=jnp.float32, precision=jax.lax.Precision.HIGHEST)

    filled = sel[:, 9:10] > 0.5
    cokb = sel[:, 8:9] > 0.5
    validb = filled & cokb
    boxes = jnp.where(validb, sel[:, 0:7], 0.0)
    scoreo = jnp.where(validb, sel[:, 7:8], 0.0)
    out_ref[...] = jnp.concatenate(
        [boxes, scoreo, validb.astype(jnp.float32), jnp.zeros((_NOUT, 7), jnp.float32)], axis=1)


def kernel(batch_box_preds, batch_cls_preds, batch_dir_preds, batch_anchors):
    b, n = batch_cls_preds.shape[0], batch_cls_preds.shape[1]
    cls_t = jnp.swapaxes(batch_cls_preds, 1, 2)
    scores = pl.pallas_call(
        _scores_body,
        grid=(b,),
        in_specs=[pl.BlockSpec((None, _NUM_CLASS, n), lambda i: (i, 0, 0))],
        out_specs=pl.BlockSpec((None, 1, n), lambda i: (i, 0, 0)),
        out_shape=jax.ShapeDtypeStruct((b, 1, n), jnp.float32),
    )(cls_t).reshape(b, n)

    sc, idx = jax.lax.top_k(scores, _NMS_PRE)
    idx2 = jnp.pad(idx.astype(jnp.int32), ((0, 0), (0, _NCAND - _NMS_PRE)))
    scp = jnp.pad(sc, ((0, 0), (0, _NCAND - _NMS_PRE)))[..., None]

    data = jnp.concatenate([batch_box_preds, batch_anchors, batch_dir_preds], axis=-1)

    out = pl.pallas_call(
        _nms_body,
        grid=(b,),
        in_specs=[
            pl.BlockSpec((None, n, 16), lambda i: (i, 0, 0)),
            pl.BlockSpec(memory_space=pltpu.SMEM),
            pl.BlockSpec((None, _NCAND, 1), lambda i: (i, 0, 0)),
        ],
        out_specs=pl.BlockSpec((None, _NOUT, 16), lambda i: (i, 0, 0)),
        out_shape=jax.ShapeDtypeStruct((b, _NOUT, 16), jnp.float32),
        scratch_shapes=[pltpu.VMEM((_NCAND, _NCAND), jnp.float32),
                        pltpu.VMEM((_NCAND, 16), jnp.float32)],
    )(data, idx2, scp)

    final_boxes = out[:, :_NMS_POST, :7]
    final_scores = out[:, :_NMS_POST, 7]
    final_labels = jnp.zeros((b, _NMS_POST), jnp.int32)
    valid = out[:, :_NMS_POST, 8] > 0.5
    return final_boxes, final_scores, final_labels, valid


# speedup vs baseline: 15.4066x; 1.2508x over previous
import jax
import jax.numpy as jnp
from jax.experimental import pallas as pl
from jax.experimental.pallas import tpu as pltpu

_NMS_PRE = 1000
_NMS_POST = 300
_IOU_TH = 0.5
_NCAND = 1024
_NOUT = 384
_NPAD = 20480
_ROWS = _NPAD // 128
_CAP = 2048
_WID = 24
_PERIOD = 3.141592653589793
_ONE_BITS = 1065353216

_HI = jax.lax.Precision.HIGHEST


def _fused_body(cls_ref, data_ref, out_ref, sup_ref, comp_ref, slot_ref, selm_ref):
    c0 = cls_ref[0]
    c1 = cls_ref[1]
    mx = jnp.maximum(c0, c1)
    e0 = jnp.exp(c0 - mx)
    e1 = jnp.exp(c1 - mx)
    s = e1 / (e0 + e1)

    sbits = jax.lax.bitcast_convert_type(s, jnp.int32)

    def bs_body(_, lohi):
        lo, hi = lohi
        mid = (lo + hi) // 2
        cnt = jnp.sum((sbits > mid).astype(jnp.int32))
        big = cnt >= _NMS_PRE
        return jnp.where(big, mid, lo), jnp.where(big, hi, mid)

    _, kth = jax.lax.fori_loop(0, 31, bs_body, (jnp.int32(-1), jnp.int32(_ONE_BITS)))
    sel = sbits >= kth
    m = jnp.sum(sel.astype(jnp.int32))
    self32 = sel.astype(jnp.float32)

    ur = jax.lax.broadcasted_iota(jnp.int32, (128, 128), 0)
    uc = jax.lax.broadcasted_iota(jnp.int32, (128, 128), 1)
    tri_incl = (ur <= uc).astype(jnp.float32)
    slot_ref[...] = jnp.dot(self32, tri_incl, preferred_element_type=jnp.float32) - 1.0
    selm_ref[...] = self32
    pvec = jax.lax.broadcasted_iota(jnp.int32, (128, 1), 0).astype(jnp.float32)
    jcol0 = jax.lax.broadcasted_iota(jnp.int32, (128, 1), 0).astype(jnp.float32)
    comp_ref[...] = jnp.zeros((_CAP, _WID), jnp.float32)

    def pack_row(r, off):
        slot_r = slot_ref[pl.ds(r, 1), :]
        sel_r = selm_ref[pl.ds(r, 1), :]
        oh = ((slot_r == pvec) & (sel_r > 0.0)).astype(jnp.float32)
        jc = jcol0 + (r * 128).astype(jnp.float32)
        d = jnp.concatenate(
            [data_ref[pl.ds(r * 128, 128), :], jc, jnp.zeros((128, _WID - 19), jnp.float32)],
            axis=1)
        part = jnp.dot(oh, d, preferred_element_type=jnp.float32, precision=_HI)
        offc = jnp.minimum(off, _CAP - 128)
        comp_ref[pl.ds(offc, 128), :] = comp_ref[pl.ds(offc, 128), :] + part
        return off + jnp.sum(sel_r).astype(jnp.int32)

    jax.lax.fori_loop(0, _ROWS, pack_row, jnp.int32(0))
    comp = comp_ref[...]

    cc0 = comp[:, 16:17]
    cc1 = comp[:, 17:18]
    cmx = jnp.maximum(cc0, cc1)
    csc = jnp.exp(cc1 - cmx) / (jnp.exp(cc0 - cmx) + jnp.exp(cc1 - cmx))
    ivec = jax.lax.broadcasted_iota(jnp.int32, (_CAP, 1), 0)
    scm = jnp.where(ivec < m, csc, -1.0)
    idxc = comp[:, 18:19]
    pairs = jnp.concatenate([scm, idxc, jnp.zeros((_CAP, 6), jnp.float32)], axis=1)
    pairs_t = pairs.T
    sc_row, idx_row = pairs_t[0:1, :], pairs_t[1:2, :]
    jrow = jax.lax.broadcasted_iota(jnp.int32, (1, _CAP), 1)
    ahead = ((sc_row > scm) | ((sc_row == scm) & (idx_row < idxc))) & (jrow < m)
    rank = jnp.dot(ahead.astype(jnp.float32), jnp.ones((_CAP, 1), jnp.float32),
                   preferred_element_type=jnp.float32)
    rank_t = jnp.concatenate([rank, jnp.zeros((_CAP, 7), jnp.float32)], axis=1).T
    rank_row = rank_t[0:1, :]

    kvec = jax.lax.broadcasted_iota(jnp.int32, (_NCAND, 1), 0)
    sel_oh = ((rank_row == kvec.astype(jnp.float32)) & (kvec < _NMS_PRE)).astype(jnp.float32)
    g = jnp.dot(sel_oh, comp, preferred_element_type=jnp.float32, precision=_HI)

    xt, yt, zt = g[:, 0:1], g[:, 1:2], g[:, 2:3]
    wt, lt, ht, rt = g[:, 3:4], g[:, 4:5], g[:, 5:6], g[:, 6:7]
    xa, ya, za = g[:, 7:8], g[:, 8:9], g[:, 9:10]
    wa, la, ha, ra = g[:, 10:11], g[:, 11:12], g[:, 12:13], g[:, 13:14]
    d0, d1 = g[:, 14:15], g[:, 15:16]
    gc0, gc1 = g[:, 16:17], g[:, 17:18]
    gmx = jnp.maximum(gc0, gc1)
    sc_cand = jnp.exp(gc1 - gmx) / (jnp.exp(gc0 - gmx) + jnp.exp(gc1 - gmx))

    diag = jnp.sqrt(la * la + wa * wa)
    xg = xt * diag + xa
    yg = yt * diag + ya
    zg = zt * ha + za
    wg = jnp.exp(wt) * wa
    lg = jnp.exp(lt) * la
    hg = jnp.exp(ht) * ha
    rg = rt + ra
    dirf = (d1 > d0).astype(jnp.float32)

    cr = jnp.abs(jnp.cos(rg))
    sr = jnp.abs(jnp.sin(rg))
    ex = (cr * wg + sr * lg) * 0.5
    ey = (sr * wg + cr * lg) * 0.5
    x1 = xg - ex
    x2 = xg + ex
    y1 = yg - ey
    y2 = yg + ey
    area = (x2 - x1) * (y2 - y1)

    scat = jnp.concatenate([x1, y1, x2, y2, area, jnp.zeros((_NCAND, 3), jnp.float32)], axis=1)
    st = scat.T
    x1r, y1r = st[0:1, :], st[1:2, :]
    x2r, y2r = st[2:3, :], st[3:4, :]
    arear = st[4:5, :]

    rb = 256
    for r0 in range(0, _NCAND, rb):
        x1b, y1b = x1[r0:r0 + rb], y1[r0:r0 + rb]
        x2b, y2b = x2[r0:r0 + rb], y2[r0:r0 + rb]
        areab = area[r0:r0 + rb]
        ix1 = jnp.maximum(x1b, x1r)
        iy1 = jnp.maximum(y1b, y1r)
        ix2 = jnp.minimum(x2b, x2r)
        iy2 = jnp.minimum(y2b, y2r)
        iw = jnp.clip(ix2 - ix1, 0.0)
        ih = jnp.clip(iy2 - iy1, 0.0)
        inter = iw * ih
        iou = inter / (areab + arear - inter + 1e-6)
        rowi = jax.lax.broadcasted_iota(jnp.int32, (rb, _NCAND), 0) + r0
        coli = jax.lax.broadcasted_iota(jnp.int32, (rb, _NCAND), 1)
        sup = (iou > _IOU_TH) & (rowi < coli) & (rowi < _NMS_PRE) & (coli < _NMS_PRE)
        sup_ref[r0:r0 + rb, :] = sup.astype(jnp.float32)

    def cond(carry):
        return carry[1]

    def body(carry):
        k, _ = carry
        sv = jnp.dot(k, sup_ref[...], preferred_element_type=jnp.float32)
        newk = jnp.where(sv > 0.0, 0.0, 1.0)
        return newk, jnp.any(newk != k)

    k0 = jnp.ones((1, _NCAND), jnp.float32)
    kfin, _ = jax.lax.while_loop(cond, body, (k0, jnp.bool_(True)))

    colv = jax.lax.broadcasted_iota(jnp.int32, (1, _NCAND), 1)
    kept = jnp.where(colv < _NMS_PRE, kfin, 0.0)

    ltr = jax.lax.broadcasted_iota(jnp.int32, (_NCAND, _NCAND), 0)
    ltc = jax.lax.broadcasted_iota(jnp.int32, (_NCAND, _NCAND), 1)
    lt_m = (ltr <= ltc).astype(jnp.float32)
    pos = jnp.dot(kept, lt_m, preferred_element_type=jnp.float32)

    orow = jax.lax.broadcasted_iota(jnp.int32, (_NOUT, 1), 0).astype(jnp.float32) + 1.0
    msel = ((pos == orow) & (kept > 0.0)).astype(jnp.float32)

    r_adj = rg - jnp.floor(rg / _PERIOD) * _PERIOD + _PERIOD * dirf
    cok = ((xg >= 0.0) & (xg <= 70.4) & (yg >= -40.0) & (yg <= 40.0)
           & (zg >= -3.0) & (zg <= 1.0)).astype(jnp.float32)
    one = jnp.ones((_NCAND, 1), jnp.float32)
    d2 = jnp.concatenate(
        [xg, yg, zg, wg, lg, hg, r_adj, sc_cand, cok, one,
         jnp.zeros((_NCAND, 6), jnp.float32)], axis=1)
    selo = jnp.dot(msel, d2, preferred_element_type=jnp.float32, precision=_HI)

    filled = selo[:, 9:10] > 0.5
    cokb = selo[:, 8:9] > 0.5
    validb = filled & cokb
    boxes = jnp.where(validb, selo[:, 0:7], 0.0)
    scoreo = jnp.where(validb, selo[:, 7:8], 0.0)
    out_ref[...] = jnp.concatenate(
        [boxes, scoreo, validb.astype(jnp.float32), jnp.zeros((_NOUT, 7), jnp.float32)], axis=1)


def kernel(batch_box_preds, batch_cls_preds, batch_dir_preds, batch_anchors):
    b, n = batch_cls_preds.shape[0], batch_cls_preds.shape[1]
    cls_t = jnp.swapaxes(batch_cls_preds, 1, 2)
    pad_c0 = jnp.zeros((b, 1, _NPAD - n), jnp.float32)
    pad_c1 = jnp.full((b, 1, _NPAD - n), -jnp.inf, jnp.float32)
    cls_p = jnp.concatenate([cls_t, jnp.concatenate([pad_c0, pad_c1], axis=1)], axis=2)
    cls4 = cls_p.reshape(b, 2, _ROWS, 128)

    data = jnp.concatenate(
        [batch_box_preds, batch_anchors, batch_dir_preds, batch_cls_preds], axis=-1)
    data = jnp.pad(data, ((0, 0), (0, _NPAD - n), (0, 0)))

    out = pl.pallas_call(
        _fused_body,
        grid=(b,),
        in_specs=[
            pl.BlockSpec((None, 2, _ROWS, 128), lambda i: (i, 0, 0, 0)),
            pl.BlockSpec((None, _NPAD, 18), lambda i: (i, 0, 0)),
        ],
        out_specs=pl.BlockSpec((None, _NOUT, 16), lambda i: (i, 0, 0)),
        out_shape=jax.ShapeDtypeStruct((b, _NOUT, 16), jnp.float32),
        scratch_shapes=[pltpu.VMEM((_NCAND, _NCAND), jnp.float32),
                        pltpu.VMEM((_CAP, _WID), jnp.float32),
                        pltpu.VMEM((_ROWS, 128), jnp.float32),
                        pltpu.VMEM((_ROWS, 128), jnp.float32)],
    )(cls4, data)

    final_boxes = out[:, :_NMS_POST, :7]
    final_scores = out[:, :_NMS_POST, 7]
    final_labels = jnp.zeros((b, _NMS_POST), jnp.int32)
    valid = out[:, :_NMS_POST, 8] > 0.5
    return final_boxes, final_scores, final_labels, valid


# speedup vs baseline: 16.6249x; 1.0791x over previous
import jax
import jax.numpy as jnp
from jax.experimental import pallas as pl
from jax.experimental.pallas import tpu as pltpu

_NMS_PRE = 1000
_NMS_POST = 300
_IOU_TH = 0.5
_NCAND = 1024
_NOUT = 384
_NPAD = 20480
_ROWS = _NPAD // 128
_CAP = 1280
_WID = 24
_PERIOD = 3.141592653589793
_ONE_BITS = 1065353216

_HI = jax.lax.Precision.HIGHEST


def _fused_body(cls_ref, data_ref, out_ref, sup_ref, comp_ref, slot_ref, selm_ref):
    c0 = cls_ref[0]
    c1 = cls_ref[1]
    mx = jnp.maximum(c0, c1)
    e0 = jnp.exp(c0 - mx)
    e1 = jnp.exp(c1 - mx)
    s = e1 / (e0 + e1)

    sbits = jax.lax.bitcast_convert_type(s, jnp.int32)

    def bs_body(_, lohi):
        lo, hi = lohi
        mid = (lo + hi) // 2
        cnt = jnp.sum((sbits > mid).astype(jnp.int32))
        big = cnt >= _NMS_PRE
        return jnp.where(big, mid, lo), jnp.where(big, hi, mid)

    _, kth = jax.lax.fori_loop(0, 31, bs_body, (jnp.int32(-1), jnp.int32(_ONE_BITS)))
    sel = sbits >= kth
    m = jnp.sum(sel.astype(jnp.int32))
    self32 = sel.astype(jnp.float32)

    ur = jax.lax.broadcasted_iota(jnp.int32, (128, 128), 0)
    uc = jax.lax.broadcasted_iota(jnp.int32, (128, 128), 1)
    tri_incl = (ur <= uc).astype(jnp.float32)
    slot_ref[...] = jnp.dot(self32, tri_incl, preferred_element_type=jnp.float32) - 1.0
    selm_ref[...] = self32
    pvec = jax.lax.broadcasted_iota(jnp.int32, (128, 1), 0).astype(jnp.float32)
    jcol0 = jax.lax.broadcasted_iota(jnp.int32, (128, 1), 0).astype(jnp.float32)
    comp_ref[...] = jnp.zeros((_CAP, _WID), jnp.float32)

    def pack_row(r, off):
        slot_r = slot_ref[pl.ds(r, 1), :]
        sel_r = selm_ref[pl.ds(r, 1), :]
        oh = ((slot_r == pvec) & (sel_r > 0.0)).astype(jnp.float32)
        jc = jcol0 + (r * 128).astype(jnp.float32)
        d = jnp.concatenate(
            [data_ref[pl.ds(r * 128, 128), :], jc, jnp.zeros((128, _WID - 19), jnp.float32)],
            axis=1)
        part = jnp.dot(oh, d, preferred_element_type=jnp.float32, precision=_HI)
        offc = jnp.minimum(off, _CAP - 128)
        comp_ref[pl.ds(offc, 128), :] = comp_ref[pl.ds(offc, 128), :] + part
        return off + jnp.sum(sel_r).astype(jnp.int32)

    jax.lax.fori_loop(0, _ROWS, pack_row, jnp.int32(0))
    comp = comp_ref[...]

    cc0 = comp[:, 16:17]
    cc1 = comp[:, 17:18]
    cmx = jnp.maximum(cc0, cc1)
    csc = jnp.exp(cc1 - cmx) / (jnp.exp(cc0 - cmx) + jnp.exp(cc1 - cmx))
    ivec = jax.lax.broadcasted_iota(jnp.int32, (_CAP, 1), 0)
    scm = jnp.where(ivec < m, csc, -1.0)
    idxc = comp[:, 18:19]
    pairs = jnp.concatenate([scm, idxc, jnp.zeros((_CAP, 6), jnp.float32)], axis=1)
    pairs_t = pairs.T
    sc_row, idx_row = pairs_t[0:1, :], pairs_t[1:2, :]
    jrow = jax.lax.broadcasted_iota(jnp.int32, (1, _CAP), 1)
    ahead = ((sc_row > scm) | ((sc_row == scm) & (idx_row < idxc))) & (jrow < m)
    rank = jnp.dot(ahead.astype(jnp.float32), jnp.ones((_CAP, 1), jnp.float32),
                   preferred_element_type=jnp.float32)
    rank_t = jnp.concatenate([rank, jnp.zeros((_CAP, 7), jnp.float32)], axis=1).T
    rank_row = rank_t[0:1, :]

    kvec = jax.lax.broadcasted_iota(jnp.int32, (_NCAND, 1), 0)
    sel_oh = ((rank_row == kvec.astype(jnp.float32)) & (kvec < _NMS_PRE)).astype(jnp.float32)
    g = jnp.dot(sel_oh, comp, preferred_element_type=jnp.float32, precision=_HI)

    xt, yt, zt = g[:, 0:1], g[:, 1:2], g[:, 2:3]
    wt, lt, ht, rt = g[:, 3:4], g[:, 4:5], g[:, 5:6], g[:, 6:7]
    xa, ya, za = g[:, 7:8], g[:, 8:9], g[:, 9:10]
    wa, la, ha, ra = g[:, 10:11], g[:, 11:12], g[:, 12:13], g[:, 13:14]
    d0, d1 = g[:, 14:15], g[:, 15:16]
    gc0, gc1 = g[:, 16:17], g[:, 17:18]
    gmx = jnp.maximum(gc0, gc1)
    sc_cand = jnp.exp(gc1 - gmx) / (jnp.exp(gc0 - gmx) + jnp.exp(gc1 - gmx))

    diag = jnp.sqrt(la * la + wa * wa)
    xg = xt * diag + xa
    yg = yt * diag + ya
    zg = zt * ha + za
    wg = jnp.exp(wt) * wa
    lg = jnp.exp(lt) * la
    hg = jnp.exp(ht) * ha
    rg = rt + ra
    dirf = (d1 > d0).astype(jnp.float32)

    cr = jnp.abs(jnp.cos(rg))
    sr = jnp.abs(jnp.sin(rg))
    ex = (cr * wg + sr * lg) * 0.5
    ey = (sr * wg + cr * lg) * 0.5
    x1 = xg - ex
    x2 = xg + ex
    y1 = yg - ey
    y2 = yg + ey
    area = (x2 - x1) * (y2 - y1)

    scat = jnp.concatenate([x1, y1, x2, y2, area, jnp.zeros((_NCAND, 3), jnp.float32)], axis=1)
    st = scat.T
    x1r, y1r = st[0:1, :], st[1:2, :]
    x2r, y2r = st[2:3, :], st[3:4, :]
    arear = st[4:5, :]

    rb = 256
    for r0 in range(0, _NCAND, rb):
        x1b, y1b = x1[r0:r0 + rb], y1[r0:r0 + rb]
        x2b, y2b = x2[r0:r0 + rb], y2[r0:r0 + rb]
        areab = area[r0:r0 + rb]
        ix1 = jnp.maximum(x1b, x1r)
        iy1 = jnp.maximum(y1b, y1r)
        ix2 = jnp.minimum(x2b, x2r)
        iy2 = jnp.minimum(y2b, y2r)
        iw = jnp.clip(ix2 - ix1, 0.0)
        ih = jnp.clip(iy2 - iy1, 0.0)
        inter = iw * ih
        iou = inter / (areab + arear - inter + 1e-6)
        rowi = jax.lax.broadcasted_iota(jnp.int32, (rb, _NCAND), 0) + r0
        coli = jax.lax.broadcasted_iota(jnp.int32, (rb, _NCAND), 1)
        sup = (iou > _IOU_TH) & (rowi < coli) & (rowi < _NMS_PRE) & (coli < _NMS_PRE)
        sup_ref[r0:r0 + rb, :] = sup.astype(jnp.float32)

    def cond(carry):
        return carry[1]

    def body(carry):
        k, _ = carry
        sv = jnp.dot(k, sup_ref[...], preferred_element_type=jnp.float32)
        newk = jnp.where(sv > 0.0, 0.0, 1.0)
        return newk, jnp.any(newk != k)

    k0 = jnp.ones((1, _NCAND), jnp.float32)
    kfin, _ = jax.lax.while_loop(cond, body, (k0, jnp.bool_(True)))

    colv = jax.lax.broadcasted_iota(jnp.int32, (1, _NCAND), 1)
    kept = jnp.where(colv < _NMS_PRE, kfin, 0.0)

    ltr = jax.lax.broadcasted_iota(jnp.int32, (_NCAND, _NCAND), 0)
    ltc = jax.lax.broadcasted_iota(jnp.int32, (_NCAND, _NCAND), 1)
    lt_m = (ltr <= ltc).astype(jnp.float32)
    pos = jnp.dot(kept, lt_m, preferred_element_type=jnp.float32)

    orow = jax.lax.broadcasted_iota(jnp.int32, (_NOUT, 1), 0).astype(jnp.float32) + 1.0
    msel = ((pos == orow) & (kept > 0.0)).astype(jnp.float32)

    r_adj = rg - jnp.floor(rg / _PERIOD) * _PERIOD + _PERIOD * dirf
    cok = ((xg >= 0.0) & (xg <= 70.4) & (yg >= -40.0) & (yg <= 40.0)
           & (zg >= -3.0) & (zg <= 1.0)).astype(jnp.float32)
    one = jnp.ones((_NCAND, 1), jnp.float32)
    d2 = jnp.concatenate(
        [xg, yg, zg, wg, lg, hg, r_adj, sc_cand, cok, one,
         jnp.zeros((_NCAND, 6), jnp.float32)], axis=1)
    selo = jnp.dot(msel, d2, preferred_element_type=jnp.float32, precision=_HI)

    filled = selo[:, 9:10] > 0.5
    cokb = selo[:, 8:9] > 0.5
    validb = filled & cokb
    boxes = jnp.where(validb, selo[:, 0:7], 0.0)
    scoreo = jnp.where(validb, selo[:, 7:8], 0.0)
    out_ref[...] = jnp.concatenate(
        [boxes, scoreo, validb.astype(jnp.float32), jnp.zeros((_NOUT, 7), jnp.float32)], axis=1)


def kernel(batch_box_preds, batch_cls_preds, batch_dir_preds, batch_anchors):
    b, n = batch_cls_preds.shape[0], batch_cls_preds.shape[1]
    cls_t = jnp.swapaxes(batch_cls_preds, 1, 2)
    pad_c0 = jnp.zeros((b, 1, _NPAD - n), jnp.float32)
    pad_c1 = jnp.full((b, 1, _NPAD - n), -jnp.inf, jnp.float32)
    cls_p = jnp.concatenate([cls_t, jnp.concatenate([pad_c0, pad_c1], axis=1)], axis=2)
    cls4 = cls_p.reshape(b, 2, _ROWS, 128)

    data = jnp.concatenate(
        [batch_box_preds, batch_anchors, batch_dir_preds, batch_cls_preds], axis=-1)
    data = jnp.pad(data, ((0, 0), (0, _NPAD - n), (0, 0)))

    out = pl.pallas_call(
        _fused_body,
        grid=(b,),
        in_specs=[
            pl.BlockSpec((None, 2, _ROWS, 128), lambda i: (i, 0, 0, 0)),
            pl.BlockSpec((None, _NPAD, 18), lambda i: (i, 0, 0)),
        ],
        out_specs=pl.BlockSpec((None, _NOUT, 16), lambda i: (i, 0, 0)),
        out_shape=jax.ShapeDtypeStruct((b, _NOUT, 16), jnp.float32),
        scratch_shapes=[pltpu.VMEM((_NCAND, _NCAND), jnp.float32),
                        pltpu.VMEM((_CAP, _WID), jnp.float32),
                        pltpu.VMEM((_ROWS, 128), jnp.float32),
                        pltpu.VMEM((_ROWS, 128), jnp.float32)],
    )(cls4, data)

    final_boxes = out[:, :_NMS_POST, :7]
    final_scores = out[:, :_NMS_POST, 7]
    final_labels = jnp.zeros((b, _NMS_POST), jnp.int32)
    valid = out[:, :_NMS_POST, 8] > 0.5
    return final_boxes, final_scores, final_labels, valid
